# baseline (speedup 1.0000x reference)
import functools

import jax
import jax.numpy as jnp
from jax import lax
from jax.experimental import pallas as pl
from jax.experimental.pallas import tpu as pltpu
from jax.experimental.pallas import tpu_sc as plsc

BATCH = 16384
FIELDS = 26
EMBED = 32
PROWS = 250000
PW = 128
NC = 2
NS = 16
NW = NC * NS
ROWS_W = BATCH // NW
PADF = 32
CHUNK = 8
NCHUNK = ROWS_W // CHUNK
NOBUF = 2
LANES = 16


def _emb_body(idx_hbm, table_hbm, out_hbm,
              idx_v, flatp_v, flats_v, gath_v, comp_v, gsem, *osems):
    wid = lax.axis_index("s") * NC + lax.axis_index("c")
    base = wid * ROWS_W

    pltpu.sync_copy(idx_hbm.at[pl.ds(base, ROWS_W), :], idx_v)

    lane = lax.iota(jnp.int32, LANES)
    chi = jnp.minimum(lane + LANES, FIELDS - 1)

    def repack_step(r, carry):
        rv = lane * 0 + r
        lo = plsc.load_gather(idx_v, [rv, lane])
        hi = plsc.load_gather(idx_v, [rv, chi])
        flatp_v[pl.ds(r * PADF, LANES)] = lo >> 2
        flatp_v[pl.ds(r * PADF + LANES, LANES)] = hi >> 2
        flats_v[pl.ds(r * PADF, LANES)] = (lo & 3) * EMBED
        flats_v[pl.ds(r * PADF + LANES, LANES)] = (hi & 3) * EMBED
        return carry

    lax.fori_loop(0, ROWS_W, repack_step, 0)

    def one_chunk(g, bo, first):

        def row_gather(k, c):
            pltpu.async_copy(
                table_hbm.at[
                    flatp_v.at[pl.ds((g * CHUNK + k) * PADF, FIELDS)]],
                gath_v.at[k], gsem)
            return c

        lax.fori_loop(0, CHUNK, row_gather, 0)
        def row_drain(k, c):
            pltpu.make_async_copy(
                table_hbm.at[pl.ds(0, FIELDS)], gath_v.at[k], gsem).wait()
            return c

        lax.fori_loop(0, CHUNK, row_drain, 0)

        @pl.when(jnp.logical_not(first))
        def _():
            pltpu.make_async_copy(
                out_hbm.at[pl.ds(0, CHUNK)], comp_v.at[bo], osems[bo]).wait()

        def extract_step(k, c):
            rb = (g * CHUNK + k) * PADF
            sv_lo = flats_v[pl.ds(rb, LANES)]
            sv_hi = flats_v[pl.ds(rb + LANES, LANES)]
            for f in range(FIELDS):
                s = sv_lo[f] if f < LANES else sv_hi[f - LANES]
                comp_v.at[bo][k, f, pl.ds(0, LANES)] = (
                    gath_v[k, f, pl.ds(s, LANES)])
                comp_v.at[bo][k, f, pl.ds(LANES, LANES)] = (
                    gath_v[k, f, pl.ds(s + LANES, LANES)])
            return c

        lax.fori_loop(0, CHUNK, extract_step, 0)

        pltpu.async_copy(
            comp_v.at[bo],
            out_hbm.at[pl.ds(base + g * CHUNK, CHUNK)],
            osems[bo])

    def super_step(i, carry):
        for b in range(NOBUF):
            one_chunk(i * NOBUF + b, b, i == 0)
        return carry

    lax.fori_loop(0, NCHUNK // NOBUF, super_step, 0)
    for b in range(NOBUF):
        pltpu.make_async_copy(
            out_hbm.at[pl.ds(0, CHUNK)], comp_v.at[b], osems[b]).wait()


@jax.jit
def kernel(x, emb_weight):
    idx = x.astype(jnp.int32)
    table = emb_weight.reshape(PROWS, PW)
    mesh = plsc.VectorSubcoreMesh(core_axis_name="c", subcore_axis_name="s")
    run = functools.partial(
        pl.kernel,
        out_type=jax.ShapeDtypeStruct((BATCH, FIELDS, EMBED), jnp.float32),
        mesh=mesh,
        scratch_types=[
            pltpu.VMEM((ROWS_W, FIELDS), jnp.int32),
            pltpu.VMEM((ROWS_W * PADF,), jnp.int32),
            pltpu.VMEM((ROWS_W * PADF,), jnp.int32),
            pltpu.VMEM((CHUNK, FIELDS, PW), jnp.float32),
            pltpu.VMEM((NOBUF, CHUNK, FIELDS, EMBED), jnp.float32),
        ] + [pltpu.SemaphoreType.DMA] * (1 + NOBUF),
        compiler_params=pltpu.CompilerParams(
            use_tc_tiling_on_sc=False, needs_layout_passes=False),
    )(_emb_body)
    return run(idx, table)
